# baseline (speedup 1.0000x reference)
import functools

import jax
import jax.numpy as jnp
from jax import lax
from jax.experimental import pallas as pl
from jax.experimental.pallas import tpu as pltpu
from jax.experimental.pallas import tpu_sc as plsc

_CHUNK = 128
_L = 16


@functools.lru_cache(maxsize=None)
def _repack(D, V, Vm):
    def body(in_ref, out_ref, sem):
        copies = [
            pltpu.make_async_copy(
                in_ref.at[d, pl.ds(0, Vm)], out_ref.at[pl.ds(d * Vm, Vm)], sem
            )
            for d in range(D)
        ]
        for c in copies:
            c.start()
        for c in copies:
            c.wait()

    return pl.pallas_call(
        body,
        in_specs=[pl.BlockSpec(memory_space=pl.ANY)],
        out_specs=pl.BlockSpec(memory_space=pl.ANY),
        scratch_shapes=[pltpu.SemaphoreType.DMA],
        out_shape=jax.ShapeDtypeStruct((D * Vm,), jnp.float32),
    )


@functools.lru_cache(maxsize=None)
def _build(B, V, D, Vm):
    info = plsc.get_sparse_core_info()
    NC, NS = info.num_cores, info.num_subcores
    NW = NC * NS
    b_per_w = B // NW
    n_chunk = b_per_w // _CHUNK
    rows_per_d = Vm // _L
    n_tail = V - Vm
    assert B % (NW * _CHUNK) == 0 and Vm % _L == 0 and D % 2 == 0
    mesh = plsc.VectorSubcoreMesh(core_axis_name="c", subcore_axis_name="s")

    @functools.partial(
        pl.kernel,
        mesh=mesh,
        out_type=jax.ShapeDtypeStruct((D, B), jnp.float32),
        scratch_types=[
            pltpu.VMEM((n_chunk, _CHUNK), jnp.int32),
            pltpu.VMEM((n_chunk, _CHUNK), jnp.int32),
            pltpu.VMEM((n_chunk, _CHUNK), jnp.int32),
            pltpu.VMEM((n_chunk, _CHUNK), jnp.int32),
            pltpu.VMEM((n_chunk, _CHUNK), jnp.int32),
            pltpu.VMEM((2, n_chunk, _CHUNK), jnp.int32),
            pltpu.VMEM((2, b_per_w, _L), jnp.float32),
            pltpu.VMEM((n_tail, D), jnp.float32),
            pltpu.VMEM((D, b_per_w), jnp.float32),
            pltpu.SemaphoreType.DMA,
        ],
        compiler_params=pltpu.CompilerParams(
            needs_layout_passes=False, use_tc_tiling_on_sc=False
        ),
    )
    def k(idx_hbm, table_hbm, tail_hbm, out_hbm, idx_v, lane_v, mrow_v,
          tidx_v, tmask_v, ridx_v, gbuf_v, tail_v, obuf_v, sem):
        wid = lax.axis_index("s") * NC + lax.axis_index("c")
        base = wid * b_per_w
        pltpu.sync_copy(tail_hbm, tail_v)
        pltpu.sync_copy(idx_hbm.at[pl.ds(wid * n_chunk, n_chunk)], idx_v)

        for j in range(n_chunk):
            for s in range(0, _CHUNK, _L):
                v = idx_v[j, pl.ds(s, _L)]
                lane_v[j, pl.ds(s, _L)] = lax.bitwise_and(v, _L - 1)
                mrow_v[j, pl.ds(s, _L)] = lax.min(
                    lax.shift_right_logical(v, 4), rows_per_d - 1
                )
                tidx_v[j, pl.ds(s, _L)] = lax.max(v - Vm, 0)
                tmask_v[j, pl.ds(s, _L)] = jnp.where(v >= Vm, 1, 0).astype(
                    jnp.int32
                )

        iota = lax.iota(jnp.int32, _L)

        def fire(d):
            slot = lax.rem(d, 2)
            off = d * rows_per_d
            for j in range(n_chunk):
                for s in range(0, _CHUNK, _L):
                    ridx_v[slot, j, pl.ds(s, _L)] = (
                        mrow_v[j, pl.ds(s, _L)] + off
                    )
            for j in range(n_chunk):
                pltpu.async_copy(
                    table_hbm.at[ridx_v.at[slot, j]],
                    gbuf_v.at[slot, pl.ds(j * _CHUNK, _CHUNK), :],
                    sem,
                )

        def consume(d):
            slot = lax.rem(d, 2)
            pltpu.make_async_copy(
                table_hbm.at[pl.ds(0, b_per_w)], gbuf_v.at[slot], sem
            ).wait()
            g2d = gbuf_v.at[slot]
            dsplat = jnp.full((_L,), 0, jnp.int32) + d
            for g in range(b_per_w // _L):
                j = g // (_CHUNK // _L)
                s = (g % (_CHUNK // _L)) * _L
                rows = iota + (g * _L)
                lanes = lane_v[j, pl.ds(s, _L)]
                vals = plsc.load_gather(g2d, [rows, lanes])
                tvals = plsc.load_gather(
                    tail_v, [tidx_v[j, pl.ds(s, _L)], dsplat]
                )
                tm = tmask_v[j, pl.ds(s, _L)]
                obuf_v[d, pl.ds(g * _L, _L)] = jnp.where(tm > 0, tvals, vals)

        def body(d, carry):
            pl.when(d < D)(lambda: fire(d))
            pl.when(d > 0)(lambda: consume(d - 1))
            return carry

        lax.fori_loop(0, D + 1, body, 0)
        pltpu.sync_copy(obuf_v, out_hbm.at[:, pl.ds(base, b_per_w)])

    return k


def kernel(user_id, embedding_weight):
    B = user_id.shape[0]
    V, D = embedding_weight.shape
    Vm = (V // _CHUNK) * _CHUNK
    idx = user_id.astype(jnp.int32).reshape(B // _CHUNK, _CHUNK)
    flat = _repack(D, V, Vm)(embedding_weight.T)
    table2 = flat.reshape((D * Vm) // _L, _L)
    tail = embedding_weight[Vm:, :]
    out_t = _build(B, V, D, Vm)(idx, table2, tail)
    return out_t.T


# speedup vs baseline: 28.3968x; 28.3968x over previous
import functools

import jax
import jax.numpy as jnp
from jax import lax
from jax.experimental import pallas as pl
from jax.experimental.pallas import tpu as pltpu
from jax.experimental.pallas import tpu_sc as plsc

_CHUNK = 128
_L = 16


@functools.lru_cache(maxsize=None)
def _repack(D, V, Vm, CB):
    n_cb = Vm // CB

    def body(in_ref, out_ref, sem):
        r = pl.program_id(0)
        cb = pl.program_id(1)
        copies = []
        for s in range(8):
            off = (r * 8 + s) * Vm + cb * CB
            copies.append(
                pltpu.make_async_copy(
                    in_ref.at[s],
                    out_ref.at[pl.ds(pl.multiple_of(off, _CHUNK), CB)],
                    sem,
                )
            )
        for c in copies:
            c.start()
        for c in copies:
            c.wait()

    return pl.pallas_call(
        body,
        grid=(D // 8, n_cb),
        in_specs=[pl.BlockSpec((8, CB), lambda r, cb: (r, cb))],
        out_specs=pl.BlockSpec(memory_space=pl.ANY),
        scratch_shapes=[pltpu.SemaphoreType.DMA],
        out_shape=jax.ShapeDtypeStruct((D * Vm,), jnp.float32),
    )


@functools.lru_cache(maxsize=None)
def _build(B, V, D, Vm):
    info = plsc.get_sparse_core_info()
    NC, NS = info.num_cores, info.num_subcores
    NW = NC * NS
    b_per_w = B // NW
    n_chunk = b_per_w // _CHUNK
    rows_per_d = Vm // _L
    n_tail = V - Vm
    assert B % (NW * _CHUNK) == 0 and Vm % _L == 0 and D % 2 == 0
    mesh = plsc.VectorSubcoreMesh(core_axis_name="c", subcore_axis_name="s")

    @functools.partial(
        pl.kernel,
        mesh=mesh,
        out_type=jax.ShapeDtypeStruct((D, B), jnp.float32),
        scratch_types=[
            pltpu.VMEM((n_chunk, _CHUNK), jnp.int32),
            pltpu.VMEM((n_chunk, _CHUNK), jnp.int32),
            pltpu.VMEM((n_chunk, _CHUNK), jnp.int32),
            pltpu.VMEM((n_chunk, _CHUNK), jnp.int32),
            pltpu.VMEM((n_chunk, _CHUNK), jnp.int32),
            pltpu.VMEM((2, n_chunk, _CHUNK), jnp.int32),
            pltpu.VMEM((2, b_per_w, _L), jnp.float32),
            pltpu.VMEM((n_tail, D), jnp.float32),
            pltpu.VMEM((D, b_per_w), jnp.float32),
            pltpu.SemaphoreType.DMA,
        ],
        compiler_params=pltpu.CompilerParams(
            needs_layout_passes=False, use_tc_tiling_on_sc=False
        ),
    )
    def k(idx_hbm, table_hbm, tail_hbm, out_hbm, idx_v, lane_v, mrow_v,
          tidx_v, tmask_v, ridx_v, gbuf_v, tail_v, obuf_v, sem):
        wid = lax.axis_index("s") * NC + lax.axis_index("c")
        base = wid * b_per_w
        pltpu.sync_copy(tail_hbm, tail_v)
        pltpu.sync_copy(idx_hbm.at[pl.ds(wid * n_chunk, n_chunk)], idx_v)

        for j in range(n_chunk):
            for s in range(0, _CHUNK, _L):
                v = idx_v[j, pl.ds(s, _L)]
                lane_v[j, pl.ds(s, _L)] = lax.bitwise_and(v, _L - 1)
                mrow_v[j, pl.ds(s, _L)] = lax.min(
                    lax.shift_right_logical(v, 4), rows_per_d - 1
                )
                tidx_v[j, pl.ds(s, _L)] = lax.max(v - Vm, 0)
                tmask_v[j, pl.ds(s, _L)] = jnp.where(v >= Vm, 1, 0).astype(
                    jnp.int32
                )

        iota = lax.iota(jnp.int32, _L)

        def fire(d):
            slot = lax.rem(d, 2)
            off = d * rows_per_d
            for j in range(n_chunk):
                for s in range(0, _CHUNK, _L):
                    ridx_v[slot, j, pl.ds(s, _L)] = (
                        mrow_v[j, pl.ds(s, _L)] + off
                    )
            for j in range(n_chunk):
                pltpu.async_copy(
                    table_hbm.at[ridx_v.at[slot, j]],
                    gbuf_v.at[slot, pl.ds(j * _CHUNK, _CHUNK), :],
                    sem,
                )

        def consume(d):
            slot = lax.rem(d, 2)
            pltpu.make_async_copy(
                table_hbm.at[pl.ds(0, b_per_w)], gbuf_v.at[slot], sem
            ).wait()
            g2d = gbuf_v.at[slot]
            dsplat = jnp.full((_L,), 0, jnp.int32) + d
            for g in range(b_per_w // _L):
                j = g // (_CHUNK // _L)
                s = (g % (_CHUNK // _L)) * _L
                rows = iota + (g * _L)
                lanes = lane_v[j, pl.ds(s, _L)]
                vals = plsc.load_gather(g2d, [rows, lanes])
                tvals = plsc.load_gather(
                    tail_v, [tidx_v[j, pl.ds(s, _L)], dsplat]
                )
                tm = tmask_v[j, pl.ds(s, _L)]
                obuf_v[d, pl.ds(g * _L, _L)] = jnp.where(tm > 0, tvals, vals)

        def body(d, carry):
            pl.when(d < D)(lambda: fire(d))
            pl.when(d > 0)(lambda: consume(d - 1))
            return carry

        lax.fori_loop(0, D + 1, body, 0)
        pltpu.sync_copy(obuf_v, out_hbm.at[:, pl.ds(base, b_per_w)])

    return k


def kernel(user_id, embedding_weight):
    B = user_id.shape[0]
    V, D = embedding_weight.shape
    Vm = (V // _CHUNK) * _CHUNK
    idx = user_id.astype(jnp.int32).reshape(B // _CHUNK, _CHUNK)
    flat = _repack(D, V, Vm, Vm // 4)(embedding_weight.T)
    table2 = flat.reshape((D * Vm) // _L, _L)
    tail = embedding_weight[Vm:, :]
    out_t = _build(B, V, D, Vm)(idx, table2, tail)
    return out_t.T


# speedup vs baseline: 28.7800x; 1.0135x over previous
import functools

import jax
import jax.numpy as jnp
from jax import lax
from jax.experimental import pallas as pl
from jax.experimental.pallas import tpu as pltpu
from jax.experimental.pallas import tpu_sc as plsc

_CHUNK = 128
_L = 16


@functools.lru_cache(maxsize=None)
def _repack(group, DS, Vm, CB):
    n_cb = Vm // CB

    def body(in_ref, out_ref, sem):
        cb = pl.program_id(0)
        copies = []
        for s in range(DS):
            off = s * Vm + cb * CB
            copies.append(
                pltpu.make_async_copy(
                    in_ref.at[s],
                    out_ref.at[pl.ds(pl.multiple_of(off, _CHUNK), CB)],
                    sem,
                )
            )
        for c in copies:
            c.start()
        for c in copies:
            c.wait()

    return pl.pallas_call(
        body,
        grid=(n_cb,),
        in_specs=[pl.BlockSpec((DS, CB), lambda cb: (group, cb))],
        out_specs=pl.BlockSpec(memory_space=pl.ANY),
        scratch_shapes=[pltpu.SemaphoreType.DMA],
        out_shape=jax.ShapeDtypeStruct((DS * Vm,), jnp.float32),
    )


@functools.lru_cache(maxsize=None)
def _build(B, V, D, Vm):
    info = plsc.get_sparse_core_info()
    NC, NS = info.num_cores, info.num_subcores
    NW = NC * NS
    b_per_w = B // NW
    n_chunk = b_per_w // _CHUNK
    rows_per_d = Vm // _L
    n_tail = V - Vm
    assert B % (NW * _CHUNK) == 0 and Vm % _L == 0 and D % 2 == 0
    mesh = plsc.VectorSubcoreMesh(core_axis_name="c", subcore_axis_name="s")

    @functools.partial(
        pl.kernel,
        mesh=mesh,
        out_type=jax.ShapeDtypeStruct((D, B), jnp.float32),
        scratch_types=[
            pltpu.VMEM((n_chunk, _CHUNK), jnp.int32),
            pltpu.VMEM((n_chunk, _CHUNK), jnp.int32),
            pltpu.VMEM((n_chunk, _CHUNK), jnp.int32),
            pltpu.VMEM((n_chunk, _CHUNK), jnp.int32),
            pltpu.VMEM((n_chunk, _CHUNK), jnp.int32),
            pltpu.VMEM((2, n_chunk, _CHUNK), jnp.int32),
            pltpu.VMEM((2, b_per_w, _L), jnp.float32),
            pltpu.VMEM((n_tail, D), jnp.float32),
            pltpu.VMEM((D, b_per_w), jnp.float32),
            pltpu.SemaphoreType.DMA,
        ],
        compiler_params=pltpu.CompilerParams(
            needs_layout_passes=False, use_tc_tiling_on_sc=False
        ),
    )
    def k(idx_hbm, table_hbm, tail_hbm, out_hbm, idx_v, lane_v, mrow_v,
          tidx_v, tmask_v, ridx_v, gbuf_v, tail_v, obuf_v, sem):
        wid = lax.axis_index("s") * NC + lax.axis_index("c")
        base = wid * b_per_w
        pltpu.sync_copy(tail_hbm, tail_v)
        pltpu.sync_copy(idx_hbm.at[pl.ds(wid * n_chunk, n_chunk)], idx_v)

        for j in range(n_chunk):
            for s in range(0, _CHUNK, _L):
                v = idx_v[j, pl.ds(s, _L)]
                lane_v[j, pl.ds(s, _L)] = lax.bitwise_and(v, _L - 1)
                mrow_v[j, pl.ds(s, _L)] = lax.min(
                    lax.shift_right_logical(v, 4), rows_per_d - 1
                )
                tidx_v[j, pl.ds(s, _L)] = lax.max(v - Vm, 0)
                tmask_v[j, pl.ds(s, _L)] = jnp.where(v >= Vm, 1, 0).astype(
                    jnp.int32
                )

        iota = lax.iota(jnp.int32, _L)

        def fire(d):
            slot = lax.rem(d, 2)
            off = d * rows_per_d
            for j in range(n_chunk):
                for s in range(0, _CHUNK, _L):
                    ridx_v[slot, j, pl.ds(s, _L)] = (
                        mrow_v[j, pl.ds(s, _L)] + off
                    )
            for j in range(n_chunk):
                pltpu.async_copy(
                    table_hbm.at[ridx_v.at[slot, j]],
                    gbuf_v.at[slot, pl.ds(j * _CHUNK, _CHUNK), :],
                    sem,
                )

        def consume(d):
            slot = lax.rem(d, 2)
            pltpu.make_async_copy(
                table_hbm.at[pl.ds(0, b_per_w)], gbuf_v.at[slot], sem
            ).wait()
            g2d = gbuf_v.at[slot]
            dsplat = jnp.full((_L,), 0, jnp.int32) + d
            for g in range(b_per_w // _L):
                j = g // (_CHUNK // _L)
                s = (g % (_CHUNK // _L)) * _L
                rows = iota + (g * _L)
                lanes = lane_v[j, pl.ds(s, _L)]
                vals = plsc.load_gather(g2d, [rows, lanes])
                tvals = plsc.load_gather(
                    tail_v, [tidx_v[j, pl.ds(s, _L)], dsplat]
                )
                tm = tmask_v[j, pl.ds(s, _L)]
                obuf_v[d, pl.ds(g * _L, _L)] = jnp.where(tm > 0, tvals, vals)

        def body(d, carry):
            pl.when(d < D)(lambda: fire(d))
            pl.when(d > 0)(lambda: consume(d - 1))
            return carry

        lax.fori_loop(0, D + 1, body, 0)
        pltpu.sync_copy(obuf_v, out_hbm.at[:, pl.ds(base, b_per_w)])

    return k


_DS = 8


def kernel(user_id, embedding_weight):
    B = user_id.shape[0]
    V, D = embedding_weight.shape
    Vm = (V // _CHUNK) * _CHUNK
    idx = user_id.astype(jnp.int32).reshape(B // _CHUNK, _CHUNK)
    table_t = embedding_weight.T
    outs = []
    for g in range(D // _DS):
        flat = _repack(g, _DS, Vm, Vm // 4)(table_t)
        table2 = flat.reshape((_DS * Vm) // _L, _L)
        tail = embedding_weight[Vm:, g * _DS:(g + 1) * _DS]
        outs.append(_build(B, V, _DS, Vm)(idx, table2, tail))
    out_t = jnp.concatenate(outs, axis=0)
    return out_t.T
